# Initial kernel scaffold; baseline (speedup 1.0000x reference)
import jax
import jax.numpy as jnp
from jax.experimental import pallas as pl


def kernel(x, mask, weight, bias):
    raise NotImplementedError("write your pallas kernel here")


# speedup vs baseline: 4.1497x; 4.1497x over previous
import functools

import jax
import jax.numpy as jnp
from jax.experimental import pallas as pl


def _conv_body(TH, W, x1_ref, x2_ref, m1_ref, m2_ref, w_ref, b_ref, out_ref):
    xblk = jnp.concatenate([x1_ref[0], x2_ref[0]], axis=1)
    xblk = xblk[:, : TH + 2, :]
    C = xblk.shape[0]
    WP = xblk.shape[2]
    xsh = jnp.concatenate(
        [xblk, jnp.roll(xblk, -1, axis=-1), jnp.roll(xblk, -2, axis=-1)],
        axis=0,
    )
    xflat = xsh.reshape(3 * C, (TH + 2) * WP)
    acc = None
    for dy in range(3):
        wdy = w_ref[dy * 3 * C : (dy + 1) * 3 * C, :]
        xsl = xflat[:, dy * WP : dy * WP + TH * WP]
        part = jax.lax.dot_general(
            wdy, xsl,
            dimension_numbers=(((0,), (0,)), ((), ())),
            preferred_element_type=jnp.float32,
        )
        acc = part if acc is None else acc + part
    Cout = acc.shape[0]
    acc = acc + b_ref[...]
    acc = acc.reshape(Cout, TH, WP)

    mblk = jnp.concatenate([m1_ref[0, 0], m2_ref[0, 0]], axis=0)
    msh = mblk + jnp.roll(mblk, -1, axis=-1) + jnp.roll(mblk, -2, axis=-1)
    cover = msh[0:TH] + msh[1 : TH + 1] + msh[2 : TH + 2]
    active = cover > 0.0
    res = jnp.where(active[None, :, :], acc, 0.0)
    out_ref[0] = res[:, :, :W]


def kernel(x, mask, weight, bias):
    B, C, H, W = x.shape
    Cout, _, KH, KW = weight.shape
    TH = 16
    WP = 256
    ntiles = H // TH
    HP = (ntiles + 1) * TH

    xp = jnp.pad(x, ((0, 0), (0, 0), (1, HP - H - 1), (1, WP - W - 1)))
    mp = jnp.pad(mask, ((0, 0), (0, 0), (1, HP - H - 1), (1, WP - W - 1)))

    wfull = weight.transpose(2, 3, 1, 0).reshape(KH * KW * C, Cout)
    b2 = bias.reshape(Cout, 1)

    grid = (B, ntiles)
    out = pl.pallas_call(
        functools.partial(_conv_body, TH, W),
        grid=grid,
        in_specs=[
            pl.BlockSpec((1, C, TH, WP), lambda b, i: (b, 0, i, 0)),
            pl.BlockSpec((1, C, TH, WP), lambda b, i: (b, 0, i + 1, 0)),
            pl.BlockSpec((1, 1, TH, WP), lambda b, i: (b, 0, i, 0)),
            pl.BlockSpec((1, 1, TH, WP), lambda b, i: (b, 0, i + 1, 0)),
            pl.BlockSpec((KH * KW * C, Cout), lambda b, i: (0, 0)),
            pl.BlockSpec((Cout, 1), lambda b, i: (0, 0)),
        ],
        out_specs=pl.BlockSpec((1, Cout, TH, W), lambda b, i: (b, 0, i, 0)),
        out_shape=jax.ShapeDtypeStruct((B, Cout, H, W), jnp.float32),
    )(xp, xp, mp, mp, wfull, b2)
    return out


# speedup vs baseline: 4.2590x; 1.0263x over previous
import functools

import jax
import jax.numpy as jnp
from jax.experimental import pallas as pl


def _conv_body(TH, W, x1_ref, x2_ref, m1_ref, m2_ref, w_ref, b_ref, out_ref):
    xblk = jnp.concatenate([x1_ref[0], x2_ref[0]], axis=1)
    xblk = xblk[:, : TH + 2, :]
    C = xblk.shape[0]
    WP = xblk.shape[2]
    xsh = jnp.concatenate(
        [xblk, jnp.roll(xblk, -1, axis=-1), jnp.roll(xblk, -2, axis=-1)],
        axis=0,
    )
    xflat = xsh.reshape(3 * C, (TH + 2) * WP)
    acc = None
    for dy in range(3):
        wdy = w_ref[dy * 3 * C : (dy + 1) * 3 * C, :]
        xsl = xflat[:, dy * WP : dy * WP + TH * WP]
        part = jax.lax.dot_general(
            wdy, xsl,
            dimension_numbers=(((0,), (0,)), ((), ())),
            preferred_element_type=jnp.float32,
        )
        acc = part if acc is None else acc + part
    Cout = acc.shape[0]
    acc = acc + b_ref[...]
    acc = acc.reshape(Cout, TH, WP)

    mblk = jnp.concatenate([m1_ref[0, 0], m2_ref[0, 0]], axis=0)
    msh = mblk + jnp.roll(mblk, -1, axis=-1) + jnp.roll(mblk, -2, axis=-1)
    cover = msh[0:TH] + msh[1 : TH + 1] + msh[2 : TH + 2]
    active = cover > 0.0
    res = jnp.where(active[None, :, :], acc, 0.0)
    out_ref[0] = res[:, :, :W]


def kernel(x, mask, weight, bias):
    B, C, H, W = x.shape
    Cout, _, KH, KW = weight.shape
    TH = 16
    WP = 256
    ntiles = H // TH
    HP = (ntiles + 1) * TH

    xp = jnp.pad(x.astype(jnp.bfloat16),
                 ((0, 0), (0, 0), (1, HP - H - 1), (1, WP - W - 1)))
    mp = jnp.pad(mask, ((0, 0), (0, 0), (1, HP - H - 1), (1, WP - W - 1)))

    wfull = weight.transpose(2, 3, 1, 0).reshape(KH * KW * C, Cout)
    wfull = wfull.astype(jnp.bfloat16)
    b2 = bias.reshape(Cout, 1)

    grid = (B, ntiles)
    out = pl.pallas_call(
        functools.partial(_conv_body, TH, W),
        grid=grid,
        in_specs=[
            pl.BlockSpec((1, C, TH, WP), lambda b, i: (b, 0, i, 0)),
            pl.BlockSpec((1, C, TH, WP), lambda b, i: (b, 0, i + 1, 0)),
            pl.BlockSpec((1, 1, TH, WP), lambda b, i: (b, 0, i, 0)),
            pl.BlockSpec((1, 1, TH, WP), lambda b, i: (b, 0, i + 1, 0)),
            pl.BlockSpec((KH * KW * C, Cout), lambda b, i: (0, 0)),
            pl.BlockSpec((Cout, 1), lambda b, i: (0, 0)),
        ],
        out_specs=pl.BlockSpec((1, Cout, TH, W), lambda b, i: (b, 0, i, 0)),
        out_shape=jax.ShapeDtypeStruct((B, Cout, H, W), jnp.float32),
    )(xp, xp, mp, mp, wfull, b2)
    return out


# speedup vs baseline: 4.5324x; 1.0642x over previous
import functools

import jax
import jax.numpy as jnp
from jax.experimental import pallas as pl


def _conv_body(TH, W, x1_ref, x2_ref, m1_ref, m2_ref, w_ref, b_ref, out_ref):
    xblk = jnp.concatenate([x1_ref[0], x2_ref[0]], axis=1)
    xblk = xblk[:, : TH + 2, :]
    C = xblk.shape[0]
    WP = xblk.shape[2]
    xsh = jnp.concatenate(
        [xblk, jnp.roll(xblk, -1, axis=-1), jnp.roll(xblk, -2, axis=-1)],
        axis=0,
    )
    xflat = xsh.reshape(3 * C, (TH + 2) * WP)
    acc = None
    for dy in range(3):
        wdy = w_ref[dy * 3 * C : (dy + 1) * 3 * C, :]
        xsl = xflat[:, dy * WP : dy * WP + TH * WP]
        part = jax.lax.dot_general(
            wdy, xsl,
            dimension_numbers=(((0,), (0,)), ((), ())),
            preferred_element_type=jnp.float32,
        )
        acc = part if acc is None else acc + part
    Cout = acc.shape[0]
    acc = acc + b_ref[...]
    acc = acc.reshape(Cout, TH, WP)

    mblk = jnp.concatenate([m1_ref[0, 0], m2_ref[0, 0]], axis=0)
    mblk = mblk[: TH + 2, :]
    msh = mblk + jnp.roll(mblk, -1, axis=-1) + jnp.roll(mblk, -2, axis=-1)
    cover = msh[0:TH] + msh[1 : TH + 1] + msh[2 : TH + 2]
    active = cover > 0.0
    res = jnp.where(active[None, :, :], acc, 0.0)
    out_ref[0] = res[:, :, :W]


def kernel(x, mask, weight, bias):
    B, C, H, W = x.shape
    Cout, _, KH, KW = weight.shape
    TH = 32
    TB = 16
    WP = 256
    ntiles = H // TH
    HP = H + TB

    xp = jnp.pad(x.astype(jnp.bfloat16),
                 ((0, 0), (0, 0), (1, HP - H - 1), (1, WP - W - 1)))
    mp = jnp.pad(mask, ((0, 0), (0, 0), (1, HP - H - 1), (1, WP - W - 1)))

    wfull = weight.transpose(2, 3, 1, 0).reshape(KH * KW * C, Cout)
    wfull = wfull.astype(jnp.bfloat16)
    b2 = bias.reshape(Cout, 1)

    grid = (B, ntiles)
    r = TH // TB
    out = pl.pallas_call(
        functools.partial(_conv_body, TH, W),
        grid=grid,
        in_specs=[
            pl.BlockSpec((1, C, TH, WP), lambda b, i: (b, 0, i, 0)),
            pl.BlockSpec((1, C, TB, WP), lambda b, i: (b, 0, r * i + r, 0)),
            pl.BlockSpec((1, 1, TH, WP), lambda b, i: (b, 0, i, 0)),
            pl.BlockSpec((1, 1, TB, WP), lambda b, i: (b, 0, r * i + r, 0)),
            pl.BlockSpec((KH * KW * C, Cout), lambda b, i: (0, 0)),
            pl.BlockSpec((Cout, 1), lambda b, i: (0, 0)),
        ],
        out_specs=pl.BlockSpec((1, Cout, TH, W), lambda b, i: (b, 0, i, 0)),
        out_shape=jax.ShapeDtypeStruct((B, Cout, H, W), jnp.float32),
    )(xp, xp, mp, mp, wfull, b2)
    return out


# speedup vs baseline: 6.0295x; 1.3303x over previous
import functools

import jax
import jax.numpy as jnp
from jax.experimental import pallas as pl
from jax.experimental.pallas import tpu as pltpu


def _conv_body(TH, W, WP, x1_ref, xn_ref, m1_ref, mn_ref, w_ref, b_ref,
               out_ref, xtop_ref, mtop_ref):
    i = pl.program_id(1)
    ntiles = pl.num_programs(1)
    C = x1_ref.shape[1]
    PADR = WP - W - 1

    xa = jnp.pad(x1_ref[0].astype(jnp.bfloat16),
                 ((0, 0), (0, 0), (1, PADR)))
    xn = jnp.pad(xn_ref[0, :, 0:1, :].astype(jnp.bfloat16),
                 ((0, 0), (0, 0), (1, PADR)))
    xn = jnp.where(i == ntiles - 1, jnp.zeros_like(xn), xn)
    xt = jnp.where(i == 0, jnp.zeros_like(xtop_ref), xtop_ref[...])
    xblk = jnp.concatenate([xt, xa, xn], axis=1)
    xtop_ref[...] = xa[:, TH - 1 : TH, :]

    xsh = jnp.concatenate(
        [xblk, jnp.roll(xblk, -1, axis=-1), jnp.roll(xblk, -2, axis=-1)],
        axis=0,
    )
    xflat = xsh.reshape(3 * C, (TH + 2) * WP)
    acc = None
    for dy in range(3):
        wdy = w_ref[dy * 3 * C : (dy + 1) * 3 * C, :]
        xsl = xflat[:, dy * WP : dy * WP + TH * WP]
        part = jax.lax.dot_general(
            wdy, xsl,
            dimension_numbers=(((0,), (0,)), ((), ())),
            preferred_element_type=jnp.float32,
        )
        acc = part if acc is None else acc + part
    Cout = acc.shape[0]
    acc = acc + b_ref[...]
    acc = acc.reshape(Cout, TH, WP)

    ma = jnp.pad(m1_ref[0, 0], ((0, 0), (1, PADR)))
    mn = jnp.pad(mn_ref[0, 0, 0:1, :], ((0, 0), (1, PADR)))
    mn = jnp.where(i == ntiles - 1, jnp.zeros_like(mn), mn)
    mt = jnp.where(i == 0, jnp.zeros_like(mtop_ref), mtop_ref[...])
    mblk = jnp.concatenate([mt[0:1], ma, mn], axis=0)
    mtop_ref[...] = ma[TH - 1 : TH, :]
    msh = mblk + jnp.roll(mblk, -1, axis=-1) + jnp.roll(mblk, -2, axis=-1)
    cover = msh[0:TH] + msh[1 : TH + 1] + msh[2 : TH + 2]
    active = cover > 0.0
    res = jnp.where(active[None, :, :], acc, 0.0)
    out_ref[0] = res[:, :, :W]


def kernel(x, mask, weight, bias):
    B, C, H, W = x.shape
    Cout, _, KH, KW = weight.shape
    TH = 32
    TB = 8
    WP = 256
    ntiles = H // TH
    nlast = H // TB - 1

    wfull = weight.transpose(2, 3, 1, 0).reshape(KH * KW * C, Cout)
    wfull = wfull.astype(jnp.bfloat16)
    b2 = bias.reshape(Cout, 1)

    r = TH // TB
    grid = (B, ntiles)
    out = pl.pallas_call(
        functools.partial(_conv_body, TH, W, WP),
        grid=grid,
        in_specs=[
            pl.BlockSpec((1, C, TH, W), lambda b, i: (b, 0, i, 0)),
            pl.BlockSpec((1, C, TB, W),
                         lambda b, i: (b, 0, jnp.minimum(r * i + r, nlast), 0)),
            pl.BlockSpec((1, 1, TH, W), lambda b, i: (b, 0, i, 0)),
            pl.BlockSpec((1, 1, TB, W),
                         lambda b, i: (b, 0, jnp.minimum(r * i + r, nlast), 0)),
            pl.BlockSpec((KH * KW * C, Cout), lambda b, i: (0, 0)),
            pl.BlockSpec((Cout, 1), lambda b, i: (0, 0)),
        ],
        out_specs=pl.BlockSpec((1, Cout, TH, W), lambda b, i: (b, 0, i, 0)),
        out_shape=jax.ShapeDtypeStruct((B, Cout, H, W), jnp.float32),
        scratch_shapes=[
            pltpu.VMEM((C, 1, WP), jnp.bfloat16),
            pltpu.VMEM((1, WP), jnp.float32),
        ],
    )(x, x, mask, mask, wfull, b2)
    return out
